# Initial kernel scaffold; baseline (speedup 1.0000x reference)
import jax
import jax.numpy as jnp
from jax.experimental import pallas as pl


def kernel(c, c_edge, c_batch, s, s_edge, s_batch, W_c0, b_c0, W_c1, b_c1, W_c2, b_c2, W_s0, b_s0, W_s1, b_s1, W_s2, b_s2, W_d, b_d, W_o, b_o):
    raise NotImplementedError("write your pallas kernel here")


# speedup vs baseline: 12.0166x; 12.0166x over previous
import functools

import jax
import jax.numpy as jnp
from jax import lax
from jax.experimental import pallas as pl
from jax.experimental.pallas import tpu as pltpu
from jax.experimental.pallas import tpu_sc as plsc

N = 10000
E = 320000
D = 128
NG = 128
FC = 64
FS = 32

NC, NS, L = 2, 16, 16
NW = NC * NS
CHUNK = 128

NPW = 320
NP = NW * NPW
EW = ((E + NW * CHUNK - 1) // (NW * CHUNK)) * CHUNK
EP = EW * NW
NCH = EW // CHUNK
RPT = NP // NS

GA = 144
GPT = GA // NS
NEG = float("-inf")

_f32 = jnp.float32
_i32 = jnp.int32


def _sc_mesh():
    return plsc.VectorSubcoreMesh(core_axis_name="c", subcore_axis_name="s")


_SC_PARAMS = pltpu.CompilerParams(use_tc_tiling_on_sc=False)


def _deg_body(cdst, sdst, ones_h, zer8_h, outc, outs, idx, ones_v, acc_c, acc_s):
    cid = lax.axis_index("c")
    sid = lax.axis_index("s")
    wid = sid * NC + cid
    pltpu.sync_copy(ones_h, ones_v)

    @pl.when(sid == 0)
    def _():
        pltpu.sync_copy(zer8_h, acc_c)
        pltpu.sync_copy(zer8_h, acc_s)

    plsc.subcore_barrier()

    def run(dst_h, acc):
        base = wid * EW

        def body(i, carry):
            pltpu.sync_copy(dst_h.at[pl.ds(base + i * CHUNK, CHUNK)], idx)
            pltpu.sync_copy(ones_v, acc.at[idx], add=True)
            return carry

        lax.fori_loop(0, NCH, body, 0)

    run(cdst, acc_c)
    run(sdst, acc_s)
    plsc.subcore_barrier()
    sl = pl.ds(sid * RPT, RPT)
    pltpu.sync_copy(acc_c.at[sl], outc.at[cid, sl])
    pltpu.sync_copy(acc_s.at[sl], outs.at[cid, sl])


def _deg_call(cdst, sdst, ones_h, zer8_h):
    return pl.kernel(
        _deg_body,
        out_type=(
            jax.ShapeDtypeStruct((NC, NP, 8), _f32),
            jax.ShapeDtypeStruct((NC, NP, 8), _f32),
        ),
        mesh=_sc_mesh(),
        compiler_params=_SC_PARAMS,
        scratch_types=[
            pltpu.VMEM((CHUNK,), _i32),
            pltpu.VMEM((CHUNK, 8), _f32),
            pltpu.VMEM_SHARED((NP, 8), _f32),
            pltpu.VMEM_SHARED((NP, 8), _f32),
        ],
    )(cdst, sdst, ones_h, zer8_h)


def _msg_body(yc_h, ys_h, csrc, cdst, ssrc, sdst, zc_h, zs_h, outc, outs,
              sidx, didx, rows_c, rows_s, acc_c, acc_s, sem):
    cid = lax.axis_index("c")
    sid = lax.axis_index("s")
    wid = sid * NC + cid

    @pl.when(sid == 0)
    def _():
        pltpu.sync_copy(zc_h, acc_c)
        pltpu.sync_copy(zs_h, acc_s)

    plsc.subcore_barrier()

    def run(y_h, src_h, dst_h, rows, acc):
        base = wid * EW

        def body(i, carry):
            off = base + i * CHUNK
            pltpu.sync_copy(src_h.at[pl.ds(off, CHUNK)], sidx)
            pltpu.sync_copy(dst_h.at[pl.ds(off, CHUNK)], didx)
            pltpu.async_copy(y_h.at[sidx], rows, sem).wait()
            pltpu.sync_copy(rows, acc.at[didx], add=True)
            return carry

        lax.fori_loop(0, NCH, body, 0)

    run(yc_h, csrc, cdst, rows_c, acc_c)
    run(ys_h, ssrc, sdst, rows_s, acc_s)
    plsc.subcore_barrier()
    sl = pl.ds(sid * RPT, RPT)
    pltpu.sync_copy(acc_c.at[sl], outc.at[cid, sl])
    pltpu.sync_copy(acc_s.at[sl], outs.at[cid, sl])


def _msg_call(yc, ys, csrc, cdst, ssrc, sdst, zc_h, zs_h):
    return pl.kernel(
        _msg_body,
        out_type=(
            jax.ShapeDtypeStruct((NC, NP, FC), _f32),
            jax.ShapeDtypeStruct((NC, NP, FS), _f32),
        ),
        mesh=_sc_mesh(),
        compiler_params=_SC_PARAMS,
        scratch_types=[
            pltpu.VMEM((CHUNK,), _i32),
            pltpu.VMEM((CHUNK,), _i32),
            pltpu.VMEM((CHUNK, FC), _f32),
            pltpu.VMEM((CHUNK, FS), _f32),
            pltpu.VMEM_SHARED((NP, FC), _f32),
            pltpu.VMEM_SHARED((NP, FS), _f32),
            pltpu.SemaphoreType.DMA,
        ],
    )(yc, ys, csrc, cdst, ssrc, sdst, zc_h, zs_h)


def _pool_one(x_h, b2d_h, wid, xbuf, bbuf, mx, sm, cn, F):
    pltpu.sync_copy(x_h.at[pl.ds(wid * NPW, NPW)], xbuf)
    pltpu.sync_copy(b2d_h.at[wid], bbuf)
    nv = F // L
    neg = jnp.full((L,), NEG, _f32)
    zero = jnp.zeros((L,), _f32)

    def init(r, carry):
        for j in range(nv):
            sl = pl.ds(j * L, L)
            mx[r, sl] = neg
            sm[r, sl] = zero
        cn[r, :] = zero
        return carry

    lax.fori_loop(0, GA, init, 0)

    one = jnp.ones((L,), _f32)

    def group(i, carry):
        bv = bbuf[pl.ds(i * L, L)]
        for k in range(L):
            g = bv[k]
            n = i * L + k
            for j in range(nv):
                sl = pl.ds(j * L, L)
                v = xbuf[n, sl]
                mx[g, sl] = jnp.maximum(mx[g, sl], v)
                sm[g, sl] = sm[g, sl] + v
            cn[g, :] = cn[g, :] + one
        return carry

    lax.fori_loop(0, NPW // L, group, 0)


def _merge(shm, part, sid, acc, rbuf, out, cid, F, is_max):
    pltpu.sync_copy(part, shm.at[sid])
    plsc.subcore_barrier()
    rows = pl.ds(sid * GPT, GPT)
    pltpu.sync_copy(shm.at[0, rows], acc)

    def body(k, carry):
        pltpu.sync_copy(shm.at[k, rows], rbuf)
        for r in range(GPT):
            for j in range(F // L):
                sl = pl.ds(j * L, L)
                if is_max:
                    acc[r, sl] = jnp.maximum(acc[r, sl], rbuf[r, sl])
                else:
                    acc[r, sl] = acc[r, sl] + rbuf[r, sl]
        return carry

    lax.fori_loop(1, NS, body, 0)
    pltpu.sync_copy(acc, out.at[cid, rows])


def _pool_body(xc_h, xs_h, cb_h, sb_h,
               omxc, osmc, ocnc, omxs, osms, ocns,
               xbc, xbs, bbc, bbs,
               mxc, smc, cnc, mxs, sms, cns,
               shc, shs, shn,
               ac, as_, an, rc, rs, rn):
    cid = lax.axis_index("c")
    sid = lax.axis_index("s")
    wid = sid * NC + cid

    _pool_one(xc_h, cb_h, wid, xbc, bbc, mxc, smc, cnc, FC)
    _pool_one(xs_h, sb_h, wid, xbs, bbs, mxs, sms, cns, FS)

    _merge(shc, mxc, sid, ac, rc, omxc, cid, FC, True)
    plsc.subcore_barrier()
    _merge(shc, smc, sid, ac, rc, osmc, cid, FC, False)
    plsc.subcore_barrier()
    _merge(shn, cnc, sid, an, rn, ocnc, cid, L, False)
    plsc.subcore_barrier()
    _merge(shs, mxs, sid, as_, rs, omxs, cid, FS, True)
    plsc.subcore_barrier()
    _merge(shs, sms, sid, as_, rs, osms, cid, FS, False)
    plsc.subcore_barrier()
    _merge(shn, cns, sid, an, rn, ocns, cid, L, False)


def _pool_call(xc3, xs3, cb2d, sb2d):
    return pl.kernel(
        _pool_body,
        out_type=(
            jax.ShapeDtypeStruct((NC, GA, FC), _f32),
            jax.ShapeDtypeStruct((NC, GA, FC), _f32),
            jax.ShapeDtypeStruct((NC, GA, L), _f32),
            jax.ShapeDtypeStruct((NC, GA, FS), _f32),
            jax.ShapeDtypeStruct((NC, GA, FS), _f32),
            jax.ShapeDtypeStruct((NC, GA, L), _f32),
        ),
        mesh=_sc_mesh(),
        compiler_params=_SC_PARAMS,
        scratch_types=[
            pltpu.VMEM((NPW, FC), _f32),
            pltpu.VMEM((NPW, FS), _f32),
            pltpu.VMEM((NPW,), _i32),
            pltpu.VMEM((NPW,), _i32),
            pltpu.VMEM((GA, FC), _f32),
            pltpu.VMEM((GA, FC), _f32),
            pltpu.VMEM((GA, L), _f32),
            pltpu.VMEM((GA, FS), _f32),
            pltpu.VMEM((GA, FS), _f32),
            pltpu.VMEM((GA, L), _f32),
            pltpu.VMEM_SHARED((NS, GA, FC), _f32),
            pltpu.VMEM_SHARED((NS, GA, FS), _f32),
            pltpu.VMEM_SHARED((NS, GA, L), _f32),
            pltpu.VMEM((GPT, FC), _f32),
            pltpu.VMEM((GPT, FS), _f32),
            pltpu.VMEM((GPT, L), _f32),
            pltpu.VMEM((GPT, FC), _f32),
            pltpu.VMEM((GPT, FS), _f32),
            pltpu.VMEM((GPT, L), _f32),
        ],
    )(xc3, xs3, cb2d, sb2d)


def _rowmask(x):
    rid = lax.broadcasted_iota(_i32, (NP, 1), 0)
    return jnp.where(rid < N, x, 0.0)


def _tc1_body(c_ref, s_ref, wc_ref, ws_ref, dgc_ref, dgs_ref,
              yc_ref, ys_ref, dc_ref, ds_ref):
    dinv_c = lax.rsqrt(dgc_ref[0, :, 0:1] + dgc_ref[1, :, 0:1] + 1.0)
    dinv_s = lax.rsqrt(dgs_ref[0, :, 0:1] + dgs_ref[1, :, 0:1] + 1.0)
    dc_ref[...] = dinv_c
    ds_ref[...] = dinv_s
    yc = jnp.dot(c_ref[...], wc_ref[...], preferred_element_type=_f32)
    ys = jnp.dot(s_ref[...], ws_ref[...], preferred_element_type=_f32)
    yc = jnp.concatenate([yc, jnp.zeros((NP - N, FC), _f32)], axis=0)
    ys = jnp.concatenate([ys, jnp.zeros((NP - N, FS), _f32)], axis=0)
    yc_ref[...] = yc * dinv_c
    ys_ref[...] = ys * dinv_s


def _tc1_call(c, s, W_c0, W_s0, degc, degs):
    return pl.pallas_call(
        _tc1_body,
        out_shape=(
            jax.ShapeDtypeStruct((NP, FC), _f32),
            jax.ShapeDtypeStruct((NP, FS), _f32),
            jax.ShapeDtypeStruct((NP, 1), _f32),
            jax.ShapeDtypeStruct((NP, 1), _f32),
        ),
    )(c, s, W_c0, W_s0, degc, degs)


def _tc2_body(last, pc_ref, yc_ref, dc_ref, bc_ref, wc_ref,
              ps_ref, ys_ref, ds_ref, bs_ref, ws_ref, oc_ref, os_ref):
    def one(p_ref, y_ref, d_ref, b_ref, w_ref, o_ref):
        d = d_ref[...]
        x = d * (p_ref[0] + p_ref[1] + y_ref[...]) + b_ref[...][None, :]
        x = _rowmask(jnp.maximum(x, 0.0))
        if last:
            o_ref[...] = x
        else:
            o_ref[...] = jnp.dot(x, w_ref[...], preferred_element_type=_f32) * d

    one(pc_ref, yc_ref, dc_ref, bc_ref, wc_ref, oc_ref)
    one(ps_ref, ys_ref, ds_ref, bs_ref, ws_ref, os_ref)


def _tc2_call(last, pc, yc, dc, bc, wc, ps, ys, ds, bs, ws):
    return pl.pallas_call(
        functools.partial(_tc2_body, last),
        out_shape=(
            jax.ShapeDtypeStruct((NP, FC), _f32),
            jax.ShapeDtypeStruct((NP, FS), _f32),
        ),
    )(pc, yc, dc, bc, wc, ps, ys, ds, bs, ws)


def _tc4_body(mxc_ref, smc_ref, cnc_ref, mxs_ref, sms_ref, cns_ref,
              wd_ref, bd_ref, wo_ref, bo_ref, out_ref, emb_ref):
    mx_c = jnp.maximum(mxc_ref[0], mxc_ref[1])[:NG]
    sm_c = (smc_ref[0] + smc_ref[1])[:NG]
    cn_c = (cnc_ref[0, :, 0:1] + cnc_ref[1, :, 0:1])[:NG]
    mx_s = jnp.maximum(mxs_ref[0], mxs_ref[1])[:NG]
    sm_s = (sms_ref[0] + sms_ref[1])[:NG]
    cn_s = (cns_ref[0, :, 0:1] + cns_ref[1, :, 0:1])[:NG]
    mean_c = sm_c / jnp.maximum(cn_c, 1.0)
    mean_s = sm_s / jnp.maximum(cn_s, 1.0)
    emb = jnp.concatenate([mx_c, mean_c, mx_s, mean_s], axis=1)
    emb_ref[...] = emb
    dense = jnp.maximum(
        jnp.dot(emb, wd_ref[...], preferred_element_type=_f32)
        + bd_ref[...][None, :], 0.0)
    out_ref[...] = (jnp.dot(dense, wo_ref[...], preferred_element_type=_f32)
                    + bo_ref[...][None, :])


def _tc4_call(mxc, smc, cnc, mxs, sms, cns, W_d, b_d, W_o, b_o):
    return pl.pallas_call(
        _tc4_body,
        out_shape=(
            jax.ShapeDtypeStruct((NG, 1), _f32),
            jax.ShapeDtypeStruct((NG, FC * 2 + FS * 2), _f32),
        ),
    )(mxc, smc, cnc, mxs, sms, cns, W_d, b_d, W_o, b_o)


def kernel(c, c_edge, c_batch, s, s_edge, s_batch,
           W_c0, b_c0, W_c1, b_c1, W_c2, b_c2,
           W_s0, b_s0, W_s1, b_s1, W_s2, b_s2,
           W_d, b_d, W_o, b_o):
    epad = jnp.full((EP - E,), N, _i32)
    csrc = jnp.concatenate([c_edge[0].astype(_i32), epad])
    cdst = jnp.concatenate([c_edge[1].astype(_i32), epad])
    ssrc = jnp.concatenate([s_edge[0].astype(_i32), epad])
    sdst = jnp.concatenate([s_edge[1].astype(_i32), epad])
    bpad = jnp.full((NP - N,), NG, _i32)
    cb2d = jnp.concatenate([c_batch.astype(_i32), bpad]).reshape(NW, NPW)
    sb2d = jnp.concatenate([s_batch.astype(_i32), bpad]).reshape(NW, NPW)

    ones8 = jnp.ones((CHUNK, 8), _f32)
    zer8 = jnp.zeros((NP, 8), _f32)
    zc = jnp.zeros((NP, FC), _f32)
    zs = jnp.zeros((NP, FS), _f32)

    degc, degs = _deg_call(cdst, sdst, ones8, zer8)
    yc, ys, dc, ds = _tc1_call(c, s, W_c0, W_s0, degc, degs)

    pc, ps = _msg_call(yc, ys, csrc, cdst, ssrc, sdst, zc, zs)
    yc, ys = _tc2_call(False, pc, yc, dc, b_c0, W_c1, ps, ys, ds, b_s0, W_s1)
    pc, ps = _msg_call(yc, ys, csrc, cdst, ssrc, sdst, zc, zs)
    yc, ys = _tc2_call(False, pc, yc, dc, b_c1, W_c2, ps, ys, ds, b_s1, W_s2)
    pc, ps = _msg_call(yc, ys, csrc, cdst, ssrc, sdst, zc, zs)
    xc3, xs3 = _tc2_call(True, pc, yc, dc, b_c2, W_c2, ps, ys, ds, b_s2, W_s2)

    mxc, smc, cnc, mxs, sms, cns = _pool_call(xc3, xs3, cb2d, sb2d)
    out, emb = _tc4_call(mxc, smc, cnc, mxs, sms, cns, W_d, b_d, W_o, b_o)
    return (out, emb)


# speedup vs baseline: 13.0380x; 1.0850x over previous
import functools

import jax
import jax.numpy as jnp
from jax import lax
from jax.experimental import pallas as pl
from jax.experimental.pallas import tpu as pltpu
from jax.experimental.pallas import tpu_sc as plsc

N = 10000
E = 320000
D = 128
NG = 128
FC = 64
FS = 32

NC, NS, L = 2, 16, 16
NW = NC * NS
CHUNK = 128

NPW = 320
NP = NW * NPW
NBUF = 4
NCH = -(-E // (NW * CHUNK * NBUF)) * NBUF
EW = NCH * CHUNK
EP = EW * NW
NOUT = NCH // NBUF
RPT = NP // NS

GA = 144
GPT = GA // NS
NEG = float("-inf")

_f32 = jnp.float32
_i32 = jnp.int32


def _sc_mesh():
    return plsc.VectorSubcoreMesh(core_axis_name="c", subcore_axis_name="s")


_SC_PARAMS = pltpu.CompilerParams(use_tc_tiling_on_sc=False)


def _deg_body(cdst, sdst, ones_h, zer8_h, outc, outs, idx, ones_v, acc_c, acc_s,
              sem):
    cid = lax.axis_index("c")
    sid = lax.axis_index("s")
    wid = sid * NC + cid
    pltpu.sync_copy(ones_h, ones_v)

    @pl.when(sid == 0)
    def _():
        pltpu.sync_copy(zer8_h, acc_c)
        pltpu.sync_copy(zer8_h, acc_s)

    plsc.subcore_barrier()

    def run(dst_h, acc):
        pltpu.sync_copy(dst_h.at[wid], idx)

        def body(i, carry):
            pltpu.async_copy(ones_v, acc.at[idx.at[i]], sem, add=True)
            return carry

        lax.fori_loop(0, NCH, body, 0)

        def drain(i, carry):
            pltpu.make_async_copy(zer8_h.at[pl.ds(0, CHUNK)], ones_v, sem).wait()
            return carry

        lax.fori_loop(0, NCH, drain, 0)

    run(cdst, acc_c)
    run(sdst, acc_s)
    plsc.subcore_barrier()
    sl = pl.ds(sid * RPT, RPT)
    pltpu.sync_copy(acc_c.at[sl], outc.at[cid, sl])
    pltpu.sync_copy(acc_s.at[sl], outs.at[cid, sl])


def _deg_call(cdst, sdst, ones_h, zer8_h):
    return pl.kernel(
        _deg_body,
        out_type=(
            jax.ShapeDtypeStruct((NC, NP, 8), _f32),
            jax.ShapeDtypeStruct((NC, NP, 8), _f32),
        ),
        mesh=_sc_mesh(),
        compiler_params=_SC_PARAMS,
        scratch_types=[
            pltpu.VMEM((NCH, CHUNK), _i32),
            pltpu.VMEM((CHUNK, 8), _f32),
            pltpu.VMEM_SHARED((NP, 8), _f32),
            pltpu.VMEM_SHARED((NP, 8), _f32),
            pltpu.SemaphoreType.DMA,
        ],
    )(cdst, sdst, ones_h, zer8_h)


def _msg_body(yc_h, ys_h, csrc, cdst, ssrc, sdst, zc_h, zs_h, outc, outs,
              sidx, didx, rc0, rc1, rc2, rc3, rs0, rs1, rs2, rs3,
              acc_c, acc_s, g0, g1, g2, g3):
    cid = lax.axis_index("c")
    sid = lax.axis_index("s")
    wid = sid * NC + cid
    gsem = (g0, g1, g2, g3)

    @pl.when(sid == 0)
    def _():
        pltpu.sync_copy(zc_h, acc_c)
        pltpu.sync_copy(zs_h, acc_s)

    plsc.subcore_barrier()

    def run(y_h, src_h, dst_h, rows, acc):
        pltpu.sync_copy(src_h.at[wid], sidx)
        pltpu.sync_copy(dst_h.at[wid], didx)
        for b in range(NBUF):
            pltpu.async_copy(y_h.at[sidx.at[b]], rows[b], gsem[b])

        def outer(i0, carry):
            for b in range(NBUF):
                i = i0 * NBUF + b
                pltpu.make_async_copy(
                    y_h.at[pl.ds(0, CHUNK)], rows[b], gsem[b]).wait()
                pltpu.sync_copy(rows[b], acc.at[didx.at[i]], add=True)

                @pl.when(i0 + 1 < NOUT)
                def _():
                    pltpu.async_copy(y_h.at[sidx.at[i + NBUF]], rows[b], gsem[b])
            return carry

        lax.fori_loop(0, NOUT, outer, 0)

    run(yc_h, csrc, cdst, (rc0, rc1, rc2, rc3), acc_c)
    run(ys_h, ssrc, sdst, (rs0, rs1, rs2, rs3), acc_s)
    plsc.subcore_barrier()
    sl = pl.ds(sid * RPT, RPT)
    pltpu.sync_copy(acc_c.at[sl], outc.at[cid, sl])
    pltpu.sync_copy(acc_s.at[sl], outs.at[cid, sl])


def _msg_call(yc, ys, csrc, cdst, ssrc, sdst, zc_h, zs_h):
    return pl.kernel(
        _msg_body,
        out_type=(
            jax.ShapeDtypeStruct((NC, NP, FC), _f32),
            jax.ShapeDtypeStruct((NC, NP, FS), _f32),
        ),
        mesh=_sc_mesh(),
        compiler_params=_SC_PARAMS,
        scratch_types=[
            pltpu.VMEM((NCH, CHUNK), _i32),
            pltpu.VMEM((NCH, CHUNK), _i32),
        ] + [pltpu.VMEM((CHUNK, FC), _f32)] * NBUF
          + [pltpu.VMEM((CHUNK, FS), _f32)] * NBUF
          + [
            pltpu.VMEM_SHARED((NP, FC), _f32),
            pltpu.VMEM_SHARED((NP, FS), _f32),
        ] + [pltpu.SemaphoreType.DMA] * NBUF,
    )(yc, ys, csrc, cdst, ssrc, sdst, zc_h, zs_h)


def _pool_one(x_h, b2d_h, wid, xbuf, bbuf, mx, sm, cn, F):
    pltpu.sync_copy(x_h.at[pl.ds(wid * NPW, NPW)], xbuf)
    pltpu.sync_copy(b2d_h.at[wid], bbuf)
    nv = F // L
    neg = jnp.full((L,), NEG, _f32)
    zero = jnp.zeros((L,), _f32)

    def init(r, carry):
        for j in range(nv):
            sl = pl.ds(j * L, L)
            mx[r, sl] = neg
            sm[r, sl] = zero
        cn[r, :] = zero
        return carry

    lax.fori_loop(0, GA, init, 0)

    one = jnp.ones((L,), _f32)

    def group(i, carry):
        bv = bbuf[pl.ds(i * L, L)]
        for k in range(L):
            g = bv[k]
            n = i * L + k
            for j in range(nv):
                sl = pl.ds(j * L, L)
                v = xbuf[n, sl]
                mx[g, sl] = jnp.maximum(mx[g, sl], v)
                sm[g, sl] = sm[g, sl] + v
            cn[g, :] = cn[g, :] + one
        return carry

    lax.fori_loop(0, NPW // L, group, 0)


def _merge(shm, part, sid, acc, rbuf, out, cid, F, is_max):
    pltpu.sync_copy(part, shm.at[sid])
    plsc.subcore_barrier()
    rows = pl.ds(sid * GPT, GPT)
    pltpu.sync_copy(shm.at[0, rows], acc)

    def body(k, carry):
        pltpu.sync_copy(shm.at[k, rows], rbuf)
        for r in range(GPT):
            for j in range(F // L):
                sl = pl.ds(j * L, L)
                if is_max:
                    acc[r, sl] = jnp.maximum(acc[r, sl], rbuf[r, sl])
                else:
                    acc[r, sl] = acc[r, sl] + rbuf[r, sl]
        return carry

    lax.fori_loop(1, NS, body, 0)
    pltpu.sync_copy(acc, out.at[cid, rows])


def _pool_body(xc_h, xs_h, cb_h, sb_h,
               omxc, osmc, ocnc, omxs, osms, ocns,
               xbc, xbs, bbc, bbs,
               mxc, smc, cnc, mxs, sms, cns,
               shc, shs, shn,
               ac, as_, an, rc, rs, rn):
    cid = lax.axis_index("c")
    sid = lax.axis_index("s")
    wid = sid * NC + cid

    _pool_one(xc_h, cb_h, wid, xbc, bbc, mxc, smc, cnc, FC)
    _pool_one(xs_h, sb_h, wid, xbs, bbs, mxs, sms, cns, FS)

    _merge(shc, mxc, sid, ac, rc, omxc, cid, FC, True)
    plsc.subcore_barrier()
    _merge(shc, smc, sid, ac, rc, osmc, cid, FC, False)
    plsc.subcore_barrier()
    _merge(shn, cnc, sid, an, rn, ocnc, cid, L, False)
    plsc.subcore_barrier()
    _merge(shs, mxs, sid, as_, rs, omxs, cid, FS, True)
    plsc.subcore_barrier()
    _merge(shs, sms, sid, as_, rs, osms, cid, FS, False)
    plsc.subcore_barrier()
    _merge(shn, cns, sid, an, rn, ocns, cid, L, False)


def _pool_call(xc3, xs3, cb2d, sb2d):
    return pl.kernel(
        _pool_body,
        out_type=(
            jax.ShapeDtypeStruct((NC, GA, FC), _f32),
            jax.ShapeDtypeStruct((NC, GA, FC), _f32),
            jax.ShapeDtypeStruct((NC, GA, L), _f32),
            jax.ShapeDtypeStruct((NC, GA, FS), _f32),
            jax.ShapeDtypeStruct((NC, GA, FS), _f32),
            jax.ShapeDtypeStruct((NC, GA, L), _f32),
        ),
        mesh=_sc_mesh(),
        compiler_params=_SC_PARAMS,
        scratch_types=[
            pltpu.VMEM((NPW, FC), _f32),
            pltpu.VMEM((NPW, FS), _f32),
            pltpu.VMEM((NPW,), _i32),
            pltpu.VMEM((NPW,), _i32),
            pltpu.VMEM((GA, FC), _f32),
            pltpu.VMEM((GA, FC), _f32),
            pltpu.VMEM((GA, L), _f32),
            pltpu.VMEM((GA, FS), _f32),
            pltpu.VMEM((GA, FS), _f32),
            pltpu.VMEM((GA, L), _f32),
            pltpu.VMEM_SHARED((NS, GA, FC), _f32),
            pltpu.VMEM_SHARED((NS, GA, FS), _f32),
            pltpu.VMEM_SHARED((NS, GA, L), _f32),
            pltpu.VMEM((GPT, FC), _f32),
            pltpu.VMEM((GPT, FS), _f32),
            pltpu.VMEM((GPT, L), _f32),
            pltpu.VMEM((GPT, FC), _f32),
            pltpu.VMEM((GPT, FS), _f32),
            pltpu.VMEM((GPT, L), _f32),
        ],
    )(xc3, xs3, cb2d, sb2d)


def _rowmask(x):
    rid = lax.broadcasted_iota(_i32, (NP, 1), 0)
    return jnp.where(rid < N, x, 0.0)


def _tc1_body(c_ref, s_ref, wc_ref, ws_ref, dgc_ref, dgs_ref,
              yc_ref, ys_ref, dc_ref, ds_ref):
    dinv_c = lax.rsqrt(dgc_ref[0, :, 0:1] + dgc_ref[1, :, 0:1] + 1.0)
    dinv_s = lax.rsqrt(dgs_ref[0, :, 0:1] + dgs_ref[1, :, 0:1] + 1.0)
    dc_ref[...] = dinv_c
    ds_ref[...] = dinv_s
    yc = jnp.dot(c_ref[...], wc_ref[...], preferred_element_type=_f32)
    ys = jnp.dot(s_ref[...], ws_ref[...], preferred_element_type=_f32)
    yc = jnp.concatenate([yc, jnp.zeros((NP - N, FC), _f32)], axis=0)
    ys = jnp.concatenate([ys, jnp.zeros((NP - N, FS), _f32)], axis=0)
    yc_ref[...] = yc * dinv_c
    ys_ref[...] = ys * dinv_s


def _tc1_call(c, s, W_c0, W_s0, degc, degs):
    return pl.pallas_call(
        _tc1_body,
        out_shape=(
            jax.ShapeDtypeStruct((NP, FC), _f32),
            jax.ShapeDtypeStruct((NP, FS), _f32),
            jax.ShapeDtypeStruct((NP, 1), _f32),
            jax.ShapeDtypeStruct((NP, 1), _f32),
        ),
    )(c, s, W_c0, W_s0, degc, degs)


def _tc2_body(last, pc_ref, yc_ref, dc_ref, bc_ref, wc_ref,
              ps_ref, ys_ref, ds_ref, bs_ref, ws_ref, oc_ref, os_ref):
    def one(p_ref, y_ref, d_ref, b_ref, w_ref, o_ref):
        d = d_ref[...]
        x = d * (p_ref[0] + p_ref[1] + y_ref[...]) + b_ref[...][None, :]
        x = _rowmask(jnp.maximum(x, 0.0))
        if last:
            o_ref[...] = x
        else:
            o_ref[...] = jnp.dot(x, w_ref[...], preferred_element_type=_f32) * d

    one(pc_ref, yc_ref, dc_ref, bc_ref, wc_ref, oc_ref)
    one(ps_ref, ys_ref, ds_ref, bs_ref, ws_ref, os_ref)


def _tc2_call(last, pc, yc, dc, bc, wc, ps, ys, ds, bs, ws):
    return pl.pallas_call(
        functools.partial(_tc2_body, last),
        out_shape=(
            jax.ShapeDtypeStruct((NP, FC), _f32),
            jax.ShapeDtypeStruct((NP, FS), _f32),
        ),
    )(pc, yc, dc, bc, wc, ps, ys, ds, bs, ws)


def _tc4_body(mxc_ref, smc_ref, cnc_ref, mxs_ref, sms_ref, cns_ref,
              wd_ref, bd_ref, wo_ref, bo_ref, out_ref, emb_ref):
    mx_c = jnp.maximum(mxc_ref[0], mxc_ref[1])[:NG]
    sm_c = (smc_ref[0] + smc_ref[1])[:NG]
    cn_c = (cnc_ref[0, :, 0:1] + cnc_ref[1, :, 0:1])[:NG]
    mx_s = jnp.maximum(mxs_ref[0], mxs_ref[1])[:NG]
    sm_s = (sms_ref[0] + sms_ref[1])[:NG]
    cn_s = (cns_ref[0, :, 0:1] + cns_ref[1, :, 0:1])[:NG]
    mean_c = sm_c / jnp.maximum(cn_c, 1.0)
    mean_s = sm_s / jnp.maximum(cn_s, 1.0)
    emb = jnp.concatenate([mx_c, mean_c, mx_s, mean_s], axis=1)
    emb_ref[...] = emb
    dense = jnp.maximum(
        jnp.dot(emb, wd_ref[...], preferred_element_type=_f32)
        + bd_ref[...][None, :], 0.0)
    out_ref[...] = (jnp.dot(dense, wo_ref[...], preferred_element_type=_f32)
                    + bo_ref[...][None, :])


def _tc4_call(mxc, smc, cnc, mxs, sms, cns, W_d, b_d, W_o, b_o):
    return pl.pallas_call(
        _tc4_body,
        out_shape=(
            jax.ShapeDtypeStruct((NG, 1), _f32),
            jax.ShapeDtypeStruct((NG, FC * 2 + FS * 2), _f32),
        ),
    )(mxc, smc, cnc, mxs, sms, cns, W_d, b_d, W_o, b_o)


def kernel(c, c_edge, c_batch, s, s_edge, s_batch,
           W_c0, b_c0, W_c1, b_c1, W_c2, b_c2,
           W_s0, b_s0, W_s1, b_s1, W_s2, b_s2,
           W_d, b_d, W_o, b_o):
    epad = jnp.full((EP - E,), N, _i32)
    def _edges(e):
        return jnp.concatenate([e.astype(_i32), epad]).reshape(NW, NCH, CHUNK)
    csrc = _edges(c_edge[0])
    cdst = _edges(c_edge[1])
    ssrc = _edges(s_edge[0])
    sdst = _edges(s_edge[1])
    bpad = jnp.full((NP - N,), NG, _i32)
    cb2d = jnp.concatenate([c_batch.astype(_i32), bpad]).reshape(NW, NPW)
    sb2d = jnp.concatenate([s_batch.astype(_i32), bpad]).reshape(NW, NPW)

    ones8 = jnp.ones((CHUNK, 8), _f32)
    zer8 = jnp.zeros((NP, 8), _f32)
    zc = jnp.zeros((NP, FC), _f32)
    zs = jnp.zeros((NP, FS), _f32)

    degc, degs = _deg_call(cdst, sdst, ones8, zer8)
    yc, ys, dc, ds = _tc1_call(c, s, W_c0, W_s0, degc, degs)

    pc, ps = _msg_call(yc, ys, csrc, cdst, ssrc, sdst, zc, zs)
    yc, ys = _tc2_call(False, pc, yc, dc, b_c0, W_c1, ps, ys, ds, b_s0, W_s1)
    pc, ps = _msg_call(yc, ys, csrc, cdst, ssrc, sdst, zc, zs)
    yc, ys = _tc2_call(False, pc, yc, dc, b_c1, W_c2, ps, ys, ds, b_s1, W_s2)
    pc, ps = _msg_call(yc, ys, csrc, cdst, ssrc, sdst, zc, zs)
    xc3, xs3 = _tc2_call(True, pc, yc, dc, b_c2, W_c2, ps, ys, ds, b_s2, W_s2)

    mxc, smc, cnc, mxs, sms, cns = _pool_call(xc3, xs3, cb2d, sb2d)
    out, emb = _tc4_call(mxc, smc, cnc, mxs, sms, cns, W_d, b_d, W_o, b_o)
    return (out, emb)
